# baseline (speedup 1.0000x reference)
import functools

import jax
import jax.numpy as jnp
from jax import lax
from jax.experimental import pallas as pl
from jax.experimental.pallas import tpu as pltpu
from jax.experimental.pallas import tpu_sc as plsc

_NU = 25000
_N = 50000
_DH = 32
_E = 800000
_IW = 0.7
_EPS = 1e-7
_MIN_NORM = 1e-15

_LANE = 256
_NSC = 16
_NB = 393
_RPS = _NB
_ROWS = _RPS * _NSC
_PAD = _ROWS * _LANE - 2 * _E
_RT = 3128


def _tan_body(w_ref, o_ref):
    w = w_ref[...]
    s = jnp.sum(w * w, axis=1, keepdims=True) - w[:, 0:1] * w[:, 0:1]
    t = jnp.sqrt(jnp.maximum(1.0 + s, _EPS))
    yn = jnp.maximum(jnp.sqrt(s), _MIN_NORM)
    th = jnp.maximum(t, 1.0 + _EPS)
    ac = jnp.log(th + jnp.sqrt(th * th - 1.0))
    res = w * (ac / yn)
    o_ref[...] = jnp.concatenate([jnp.zeros_like(t), res[:, 1:]], axis=1)


def _tangent(w):
    rb = 1000
    n = w.shape[0]
    return pl.pallas_call(
        _tan_body,
        grid=(n // rb,),
        in_specs=[pl.BlockSpec((rb, _DH), lambda i: (i, 0))],
        out_specs=pl.BlockSpec((rb, _DH), lambda i: (i, 0)),
        out_shape=jax.ShapeDtypeStruct((n, _DH), jnp.float32),
    )(w)


def _post_body(a_ref, o_ref):
    a = a_ref[...]
    s = jnp.sum(a * a, axis=1, keepdims=True) - a[:, 0:1] * a[:, 0:1]
    xn = jnp.maximum(jnp.sqrt(s), _MIN_NORM)
    e = jnp.exp(xn)
    coef = (0.5 * (e - 1.0 / e)) / xn
    res = a * coef
    t = jnp.sqrt(jnp.maximum(1.0 + coef * coef * s, _EPS))
    o_ref[...] = jnp.concatenate([t, res[:, 1:]], axis=1)


def _post(acc):
    rb = 1000
    return pl.pallas_call(
        _post_body,
        grid=(_N // rb,),
        in_specs=[pl.BlockSpec((rb, 2 * _DH), lambda i: (i, 0))],
        out_specs=pl.BlockSpec((rb, 2 * _DH), lambda i: (i, 0)),
        out_shape=jax.ShapeDtypeStruct((_N, 2 * _DH), jnp.float32),
    )(acc)


_mesh = plsc.VectorSubcoreMesh(core_axis_name="c", subcore_axis_name="s")


@functools.partial(
    pl.kernel,
    out_type=(
        jax.ShapeDtypeStruct((2 * _N, _DH), jnp.float32),
        jax.ShapeDtypeStruct((2 * _N, _DH), jnp.float32),
    ),
    mesh=_mesh,
    compiler_params=pltpu.CompilerParams(use_tc_tiling_on_sc=False),
    scratch_types=[
        pltpu.VMEM_SHARED((_N, _DH), jnp.float32),
        pltpu.VMEM((4, _LANE), jnp.int32),
        pltpu.VMEM((4, _LANE), jnp.int32),
        pltpu.VMEM((4, _LANE), jnp.float32),
        pltpu.VMEM((2, _LANE, _DH), jnp.float32),
        pltpu.SemaphoreType.DMA,
        pltpu.SemaphoreType.DMA,
        pltpu.SemaphoreType.DMA,
    ],
)
def _sc_spmm(emb_hbm, src_hbm, dst_hbm, val_hbm, cur_hbm, acc_hbm,
             acc_sp, src_v, dst_v, val_v, rows_v, isem, gsem, ssem):
    c = lax.axis_index("c")
    s = lax.axis_index("s")
    coff = c * _N

    zero16 = jnp.zeros((16,), jnp.float32)

    def zb(i, carry):
        rows_v[0, i, 0:16] = zero16
        rows_v[0, i, 16:32] = zero16
        return carry

    lax.fori_loop(0, 128, zb, 0)
    base0 = jnp.minimum(s * _RT, _N - _RT)
    for m in range(25):
        st = jnp.minimum(base0 + m * 128, _N - 128)
        pltpu.sync_copy(rows_v.at[0, pl.ds(0, 128)],
                        acc_sp.at[pl.ds(st, 128)])
    plsc.subcore_barrier()

    row0 = s * _RPS

    def fire_idx(kb, q):
        r = (row0 + kb) * _LANE
        pltpu.async_copy(src_hbm.at[pl.ds(r, _LANE)], src_v.at[q], isem)
        pltpu.async_copy(dst_hbm.at[pl.ds(r, _LANE)], dst_v.at[q], isem)
        pltpu.async_copy(val_hbm.at[pl.ds(r, _LANE)], val_v.at[q], isem)

    def wait_idx(q):
        r0 = row0 * _LANE
        pltpu.make_async_copy(src_hbm.at[pl.ds(r0, _LANE)], src_v.at[q],
                              isem).wait()
        pltpu.make_async_copy(dst_hbm.at[pl.ds(r0, _LANE)], dst_v.at[q],
                              isem).wait()
        pltpu.make_async_copy(val_hbm.at[pl.ds(r0, _LANE)], val_v.at[q],
                              isem).wait()

    def offset_idx(q):
        for h in range(_LANE // 16):
            sl = pl.ds(h * 16, 16)
            src_v[q, sl] = src_v[q, sl] + coff

    def scale(p, q):
        @plsc.parallel_loop(0, _LANE // 16, unroll=2)
        def _scale(g):
            vals16 = val_v[q, pl.ds(g * 16, 16)]
            for l in range(16):
                v = vals16[l]
                e = g * 16 + l
                rows_v[p, e, 0:16] = rows_v[p, e, 0:16] * v
                rows_v[p, e, 16:32] = rows_v[p, e, 16:32] * v

    for layer in range(2):
        src_tbl = emb_hbm if layer == 0 else cur_hbm

        def fire_gather(p, q):
            pltpu.async_copy(src_tbl.at[src_v.at[q]], rows_v.at[p], gsem)

        def wait_gather(p, q):
            pltpu.make_async_copy(src_tbl.at[src_v.at[q]], rows_v.at[p],
                                  gsem).wait()

        def fire_scatter(p, q):
            pltpu.async_copy(rows_v.at[p], acc_sp.at[dst_v.at[q]], ssem,
                             add=True)

        def wait_scatter(p, q):
            pltpu.make_async_copy(rows_v.at[p], acc_sp.at[dst_v.at[q]],
                                  ssem).wait()

        fire_idx(0, 0)
        wait_idx(0)
        offset_idx(0)
        fire_idx(1, 1)
        fire_idx(2, 2)
        fire_gather(0, 0)

        def block_body(kb, carry):
            for par in range(4):
                @pl.when(kb % 4 == par)
                def _():
                    p = par % 2
                    m = par
                    wait_gather(p, m)
                    @pl.when(kb + 1 < _NB)
                    def _():
                        wait_idx((m + 1) % 4)
                        offset_idx((m + 1) % 4)
                    @pl.when(kb >= 1)
                    def _():
                        wait_scatter(1 - p, (m + 3) % 4)
                    @pl.when(kb + 3 < _NB)
                    def _():
                        fire_idx(kb + 3, (m + 3) % 4)
                    @pl.when(kb + 1 < _NB)
                    def _():
                        fire_gather(1 - p, (m + 1) % 4)
                    fire_scatter(p, m)
            return carry

        lax.fori_loop(0, _NB, block_body, 0)
        wait_scatter((_NB - 1) % 2, (_NB - 1) % 4)

        plsc.subcore_barrier()
        out_tbl = cur_hbm if layer == 0 else acc_hbm
        pltpu.sync_copy(acc_sp.at[pl.ds(base0, _RT)],
                        out_tbl.at[pl.ds(coff + base0, _RT)])
        plsc.subcore_barrier()
        del out_tbl


def kernel(emb_weight, user_social_feature, adj_uv_indices, adj_uv_values,
           adj_uu_indices, adj_uu_values):
    xt = _tangent(emb_weight)
    ut = _tangent(user_social_feature)
    emb_tbl = jnp.concatenate([xt, ut, xt[_NU:]], axis=0)

    zpi = jnp.zeros((_PAD,), jnp.int32)
    src = jnp.concatenate([adj_uv_indices[1], adj_uu_indices[1], zpi])
    dst = jnp.concatenate([adj_uv_indices[0], adj_uu_indices[0], zpi])
    val = jnp.concatenate([_IW * adj_uv_values, (1.0 - _IW) * adj_uu_values,
                           jnp.zeros((_PAD,), jnp.float32)])

    _cur, acc = _sc_spmm(emb_tbl, src, dst, val)
    accf = jnp.concatenate([acc[:_N], acc[_N:]], axis=1)
    return _post(accf)


# speedup vs baseline: 1.0039x; 1.0039x over previous
import functools

import jax
import jax.numpy as jnp
from jax import lax
from jax.experimental import pallas as pl
from jax.experimental.pallas import tpu as pltpu
from jax.experimental.pallas import tpu_sc as plsc

_NU = 25000
_N = 50000
_DH = 32
_E = 800000
_IW = 0.7
_EPS = 1e-7
_MIN_NORM = 1e-15

_LANE = 256
_NSC = 16
_NB = 393
_RPS = _NB
_ROWS = _RPS * _NSC
_PAD = _ROWS * _LANE - 2 * _E
_RT = 3128


def _tan_body(w_ref, o_ref):
    w = w_ref[...]
    s = jnp.sum(w * w, axis=1, keepdims=True) - w[:, 0:1] * w[:, 0:1]
    t = jnp.sqrt(jnp.maximum(1.0 + s, _EPS))
    yn = jnp.maximum(jnp.sqrt(s), _MIN_NORM)
    th = jnp.maximum(t, 1.0 + _EPS)
    ac = jnp.log(th + jnp.sqrt(th * th - 1.0))
    res = w * (ac / yn)
    o_ref[...] = jnp.concatenate([jnp.zeros_like(t), res[:, 1:]], axis=1)


def _tangent(w):
    rb = 1000
    n = w.shape[0]
    return pl.pallas_call(
        _tan_body,
        grid=(n // rb,),
        in_specs=[pl.BlockSpec((rb, _DH), lambda i: (i, 0))],
        out_specs=pl.BlockSpec((rb, _DH), lambda i: (i, 0)),
        out_shape=jax.ShapeDtypeStruct((n, _DH), jnp.float32),
    )(w)


def _post_body(a_ref, o_ref):
    a = a_ref[...]
    s = jnp.sum(a * a, axis=1, keepdims=True) - a[:, 0:1] * a[:, 0:1]
    xn = jnp.maximum(jnp.sqrt(s), _MIN_NORM)
    e = jnp.exp(xn)
    coef = (0.5 * (e - 1.0 / e)) / xn
    res = a * coef
    t = jnp.sqrt(jnp.maximum(1.0 + coef * coef * s, _EPS))
    o_ref[...] = jnp.concatenate([t, res[:, 1:]], axis=1)


def _post(acc):
    rb = 1000
    return pl.pallas_call(
        _post_body,
        grid=(_N // rb,),
        in_specs=[pl.BlockSpec((rb, 2 * _DH), lambda i: (i, 0))],
        out_specs=pl.BlockSpec((rb, 2 * _DH), lambda i: (i, 0)),
        out_shape=jax.ShapeDtypeStruct((_N, 2 * _DH), jnp.float32),
    )(acc)


_mesh = plsc.VectorSubcoreMesh(core_axis_name="c", subcore_axis_name="s")


@functools.partial(
    pl.kernel,
    out_type=(
        jax.ShapeDtypeStruct((2 * _N, _DH), jnp.float32),
        jax.ShapeDtypeStruct((2 * _N, _DH), jnp.float32),
    ),
    mesh=_mesh,
    compiler_params=pltpu.CompilerParams(use_tc_tiling_on_sc=False),
    scratch_types=[
        pltpu.VMEM_SHARED((_N, _DH), jnp.float32),
        pltpu.VMEM((6, _LANE), jnp.int32),
        pltpu.VMEM((6, _LANE), jnp.int32),
        pltpu.VMEM((6, _LANE), jnp.float32),
        pltpu.VMEM((3, _LANE, _DH), jnp.float32),
        pltpu.SemaphoreType.DMA,
        pltpu.SemaphoreType.DMA,
        pltpu.SemaphoreType.DMA,
    ],
)
def _sc_spmm(emb_hbm, src_hbm, dst_hbm, val_hbm, cur_hbm, acc_hbm,
             acc_sp, src_v, dst_v, val_v, rows_v, isem, gsem, ssem):
    c = lax.axis_index("c")
    s = lax.axis_index("s")
    coff = c * _N

    zero16 = jnp.zeros((16,), jnp.float32)

    def zb(i, carry):
        rows_v[0, i, 0:16] = zero16
        rows_v[0, i, 16:32] = zero16
        return carry

    lax.fori_loop(0, 128, zb, 0)
    base0 = jnp.minimum(s * _RT, _N - _RT)
    for m in range(25):
        st = jnp.minimum(base0 + m * 128, _N - 128)
        pltpu.sync_copy(rows_v.at[0, pl.ds(0, 128)],
                        acc_sp.at[pl.ds(st, 128)])
    plsc.subcore_barrier()

    row0 = s * _RPS

    def fire_idx(kb, q):
        r = (row0 + kb) * _LANE
        pltpu.async_copy(src_hbm.at[pl.ds(r, _LANE)], src_v.at[q], isem)
        pltpu.async_copy(dst_hbm.at[pl.ds(r, _LANE)], dst_v.at[q], isem)
        pltpu.async_copy(val_hbm.at[pl.ds(r, _LANE)], val_v.at[q], isem)

    def wait_idx(q):
        r0 = row0 * _LANE
        pltpu.make_async_copy(src_hbm.at[pl.ds(r0, _LANE)], src_v.at[q],
                              isem).wait()
        pltpu.make_async_copy(dst_hbm.at[pl.ds(r0, _LANE)], dst_v.at[q],
                              isem).wait()
        pltpu.make_async_copy(val_hbm.at[pl.ds(r0, _LANE)], val_v.at[q],
                              isem).wait()

    def offset_idx(q):
        for h in range(_LANE // 16):
            sl = pl.ds(h * 16, 16)
            src_v[q, sl] = src_v[q, sl] + coff

    def scale(p, q):
        @plsc.parallel_loop(0, _LANE // 16, unroll=2)
        def _scale(g):
            vals16 = val_v[q, pl.ds(g * 16, 16)]
            for l in range(16):
                v = vals16[l]
                e = g * 16 + l
                rows_v[p, e, 0:16] = rows_v[p, e, 0:16] * v
                rows_v[p, e, 16:32] = rows_v[p, e, 16:32] * v

    for layer in range(2):
        src_tbl = emb_hbm if layer == 0 else cur_hbm

        def fire_gather(p, q):
            pltpu.async_copy(src_tbl.at[src_v.at[q]], rows_v.at[p], gsem)

        def wait_gather(p, q):
            pltpu.make_async_copy(src_tbl.at[src_v.at[q]], rows_v.at[p],
                                  gsem).wait()

        def fire_scatter(p, q):
            pltpu.async_copy(rows_v.at[p], acc_sp.at[dst_v.at[q]], ssem,
                             add=True)

        def wait_scatter(p, q):
            pltpu.make_async_copy(rows_v.at[p], acc_sp.at[dst_v.at[q]],
                                  ssem).wait()

        fire_idx(0, 0)
        wait_idx(0)
        offset_idx(0)
        fire_idx(1, 1)
        fire_idx(2, 2)
        fire_gather(0, 0)

        def block_body(kb, carry):
            for par in range(4):
                @pl.when(kb % 4 == par)
                def _():
                    p = par % 2
                    m = par
                    wait_gather(p, m)
                    @pl.when(kb + 1 < _NB)
                    def _():
                        wait_idx((m + 1) % 4)
                        offset_idx((m + 1) % 4)
                    @pl.when(kb >= 1)
                    def _():
                        wait_scatter(1 - p, (m + 3) % 4)
                    @pl.when(kb + 3 < _NB)
                    def _():
                        fire_idx(kb + 3, (m + 3) % 4)
                    @pl.when(kb + 1 < _NB)
                    def _():
                        fire_gather(1 - p, (m + 1) % 4)
                    fire_scatter(p, m)
            return carry

        lax.fori_loop(0, _NB, block_body, 0)
        wait_scatter((_NB - 1) % 2, (_NB - 1) % 4)

        plsc.subcore_barrier()
        out_tbl = cur_hbm if layer == 0 else acc_hbm
        pltpu.sync_copy(acc_sp.at[pl.ds(base0, _RT)],
                        out_tbl.at[pl.ds(coff + base0, _RT)])
        plsc.subcore_barrier()
        del out_tbl


def kernel(emb_weight, user_social_feature, adj_uv_indices, adj_uv_values,
           adj_uu_indices, adj_uu_values):
    xt = _tangent(emb_weight)
    ut = _tangent(user_social_feature)
    emb_tbl = jnp.concatenate([xt, ut, xt[_NU:]], axis=0)

    zpi = jnp.zeros((_PAD,), jnp.int32)
    src = jnp.concatenate([adj_uv_indices[1], adj_uu_indices[1], zpi])
    dst = jnp.concatenate([adj_uv_indices[0], adj_uu_indices[0], zpi])
    val = jnp.concatenate([_IW * adj_uv_values, (1.0 - _IW) * adj_uu_values,
                           jnp.zeros((_PAD,), jnp.float32)])

    _cur, acc = _sc_spmm(emb_tbl, src, dst, val)
    accf = jnp.concatenate([acc[:_N], acc[_N:]], axis=1)
    return _post(accf)


# speedup vs baseline: 1.8852x; 1.8779x over previous
import functools

import jax
import jax.numpy as jnp
from jax import lax
from jax.experimental import pallas as pl
from jax.experimental.pallas import tpu as pltpu
from jax.experimental.pallas import tpu_sc as plsc

_NU = 25000
_N = 50000
_DH = 32
_E = 800000
_IW = 0.7
_EPS = 1e-7
_MIN_NORM = 1e-15

_LANE = 256
_NSC = 16
_EROWS = _E // _LANE
_RBASE = _EROWS // _NSC
_RT = 3128


def _lane_groups(width):
    li = lax.broadcasted_iota(jnp.int32, (128, 128), 0)
    lj = lax.broadcasted_iota(jnp.int32, (128, 128), 1)
    blk = (li // width == lj // width).astype(jnp.float32)
    lane = lax.broadcasted_iota(jnp.int32, (1, 128), 1)
    m0 = (lane % width != 0).astype(jnp.float32)
    return blk, m0


def _pre_body(ew_ref, us_ref, o_ref):
    w = jnp.concatenate(
        [ew_ref[...], us_ref[...], ew_ref[pl.ds(_NU // 4, _NU // 4), :]],
        axis=0)
    blk, m0 = _lane_groups(32)
    y2 = w * w * m0
    s = jax.lax.dot_general(y2, blk, (((1,), (0,)), ((), ())),
                            preferred_element_type=jnp.float32)
    t = jnp.sqrt(jnp.maximum(1.0 + s, _EPS))
    yn = jnp.maximum(jnp.sqrt(s), _MIN_NORM)
    th = jnp.maximum(t, 1.0 + _EPS)
    ac = jnp.log(th + jnp.sqrt(th * th - 1.0))
    o_ref[...] = w * (ac / yn) * m0


def _pre(emb_weight, user_social_feature):
    ew = emb_weight.reshape(_N // 4, 128)
    us = user_social_feature.reshape(_NU // 4, 128)
    out = pl.pallas_call(
        _pre_body,
        out_shape=jax.ShapeDtypeStruct((2 * _N // 4, 128), jnp.float32),
    )(ew, us)
    return out.reshape(2 * _N, _DH)


def _post_body(a_ref, o_ref):
    a = a_ref[...]
    blk, m0 = _lane_groups(64)
    s = jax.lax.dot_general(a * a * m0, blk, (((1,), (0,)), ((), ())),
                            preferred_element_type=jnp.float32)
    xn = jnp.maximum(jnp.sqrt(s), _MIN_NORM)
    e = jnp.exp(xn)
    coef = (0.5 * (e - 1.0 / e)) / xn
    res = a * coef * m0
    t = jnp.sqrt(jnp.maximum(1.0 + coef * coef * s, _EPS))
    o_ref[...] = res + t * (1.0 - m0)


def _post(acc64):
    out = pl.pallas_call(
        _post_body,
        out_shape=jax.ShapeDtypeStruct((_N // 2, 128), jnp.float32),
    )(acc64.reshape(_N // 2, 128))
    return out.reshape(_N, 2 * _DH)


_mesh = plsc.VectorSubcoreMesh(core_axis_name="c", subcore_axis_name="s")


@functools.partial(
    pl.kernel,
    out_type=(
        jax.ShapeDtypeStruct((2 * _N, _DH), jnp.float32),
        jax.ShapeDtypeStruct((_N, 2 * _DH), jnp.float32),
    ),
    mesh=_mesh,
    compiler_params=pltpu.CompilerParams(use_tc_tiling_on_sc=False),
    scratch_types=[
        pltpu.VMEM_SHARED((_N, _DH), jnp.float32),
        pltpu.VMEM((6, _LANE), jnp.int32),
        pltpu.VMEM((6, _LANE), jnp.int32),
        pltpu.VMEM((6, _LANE), jnp.float32),
        pltpu.VMEM((3, _LANE, _DH), jnp.float32),
        pltpu.SemaphoreType.DMA,
        pltpu.SemaphoreType.DMA,
        pltpu.SemaphoreType.DMA,
    ],
)
def _sc_spmm(emb_hbm, uvi_hbm, uvv_hbm, uui_hbm, uuv_hbm, cur_hbm, acc_hbm,
             acc_sp, src_v, dst_v, val_v, rows_v, isem, gsem, ssem):
    c = lax.axis_index("c")
    s = lax.axis_index("s")
    coff = c * _N

    rem = _EROWS - _NSC * _RBASE
    na = _RBASE + jnp.where(s < rem, 1, 0)
    uv0 = s * _RBASE + jnp.minimum(s, rem)
    nblocks = 2 * na

    zero16 = jnp.zeros((16,), jnp.float32)

    def zb(i, carry):
        rows_v[0, i, 0:16] = zero16
        rows_v[0, i, 16:32] = zero16
        return carry

    lax.fori_loop(0, 128, zb, 0)
    base0 = jnp.minimum(s * _RT, _N - _RT)
    for m in range(25):
        st = jnp.minimum(base0 + m * 128, _N - 128)
        pltpu.sync_copy(rows_v.at[0, pl.ds(0, 128)],
                        acc_sp.at[pl.ds(st, 128)])
    plsc.subcore_barrier()

    def fire_idx(kb, q):
        @pl.when(kb < na)
        def _():
            r = uv0 + kb
            pltpu.async_copy(uvi_hbm.at[1, r], src_v.at[q], isem)
            pltpu.async_copy(uvi_hbm.at[0, r], dst_v.at[q], isem)
            pltpu.async_copy(uvv_hbm.at[r], val_v.at[q], isem)

        @pl.when(kb >= na)
        def _():
            r = uv0 + kb - na
            pltpu.async_copy(uui_hbm.at[1, r], src_v.at[q], isem)
            pltpu.async_copy(uui_hbm.at[0, r], dst_v.at[q], isem)
            pltpu.async_copy(uuv_hbm.at[r], val_v.at[q], isem)

    def wait_idx(q):
        pltpu.make_async_copy(uvi_hbm.at[1, 0], src_v.at[q], isem).wait()
        pltpu.make_async_copy(uvi_hbm.at[0, 0], dst_v.at[q], isem).wait()
        pltpu.make_async_copy(uvv_hbm.at[0], val_v.at[q], isem).wait()

    def offset_idx(q, kb):
        w = jnp.where(kb < na, _IW, 1.0 - _IW).astype(jnp.float32)
        for h in range(_LANE // 16):
            sl = pl.ds(h * 16, 16)
            src_v[q, sl] = src_v[q, sl] + coff
            val_v[q, sl] = val_v[q, sl] * w

    def scale(p, q):
        @plsc.parallel_loop(0, _LANE // 16, unroll=2)
        def _scale(g):
            vals16 = val_v[q, pl.ds(g * 16, 16)]
            for l in range(16):
                v = vals16[l]
                e = g * 16 + l
                rows_v[p, e, 0:16] = rows_v[p, e, 0:16] * v
                rows_v[p, e, 16:32] = rows_v[p, e, 16:32] * v

    for layer in range(2):
        src_tbl = emb_hbm if layer == 0 else cur_hbm

        def fire_gather(p, q):
            pltpu.async_copy(src_tbl.at[src_v.at[q]], rows_v.at[p], gsem)

        def wait_gather(p, q):
            pltpu.make_async_copy(src_tbl.at[src_v.at[q]], rows_v.at[p],
                                  gsem).wait()

        def fire_scatter(p, q):
            pltpu.async_copy(rows_v.at[p], acc_sp.at[dst_v.at[q]], ssem,
                             add=True)

        def wait_scatter(p, q):
            pltpu.make_async_copy(rows_v.at[p], acc_sp.at[dst_v.at[q]],
                                  ssem).wait()

        fire_idx(0, 0)
        wait_idx(0)
        offset_idx(0, 0)
        for q in range(1, 5):
            fire_idx(q, q)
        fire_gather(0, 0)
        wait_idx(1)
        offset_idx(1, 1)
        fire_gather(1, 1)

        def block_body(kb, carry):
            for par in range(6):
                @pl.when(kb % 6 == par)
                def _():
                    p = par % 3
                    m = par
                    wait_gather(p, m)
                    @pl.when(kb >= 1)
                    def _():
                        wait_scatter((par + 2) % 3, (m + 5) % 6)
                    @pl.when(kb + 5 < nblocks)
                    def _():
                        fire_idx(kb + 5, (m + 5) % 6)
                    @pl.when(kb + 2 < nblocks)
                    def _():
                        wait_idx((m + 2) % 6)
                        offset_idx((m + 2) % 6, kb + 2)
                        fire_gather((par + 2) % 3, (m + 2) % 6)
                    scale(p, m)
                    fire_scatter(p, m)
            return carry

        lax.fori_loop(0, nblocks, block_body, 0)
        wait_scatter((nblocks - 1) % 3, (nblocks - 1) % 6)

        plsc.subcore_barrier()
        if layer == 0:
            pltpu.sync_copy(acc_sp.at[pl.ds(base0, _RT)],
                            cur_hbm.at[pl.ds(coff + base0, _RT)])
        else:
            pltpu.sync_copy(acc_sp.at[pl.ds(base0, _RT)],
                            acc_hbm.at[pl.ds(base0, _RT), pl.ds(c * _DH, _DH)])
        plsc.subcore_barrier()


def kernel(emb_weight, user_social_feature, adj_uv_indices, adj_uv_values,
           adj_uu_indices, adj_uu_values):
    tbl = _pre(emb_weight, user_social_feature)
    uvi = adj_uv_indices.reshape(2, _EROWS, _LANE)
    uvv = adj_uv_values.reshape(_EROWS, _LANE)
    uui = adj_uu_indices.reshape(2, _EROWS, _LANE)
    uuv = adj_uu_values.reshape(_EROWS, _LANE)
    _cur, acc64 = _sc_spmm(tbl, uvi, uvv, uui, uuv)
    return _post(acc64)
